# Initial kernel scaffold; baseline (speedup 1.0000x reference)
import jax
import jax.numpy as jnp
from jax.experimental import pallas as pl


def kernel(x, edge_index, W0, b0, g0, be0, W1, b1, g1, be1, W2, b2, g2, be2):
    raise NotImplementedError("write your pallas kernel here")


# speedup vs baseline: 1.9879x; 1.9879x over previous
import functools

import jax
import jax.numpy as jnp
from jax import lax
from jax.experimental import pallas as pl
from jax.experimental.pallas import tpu as pltpu
from jax.experimental.pallas import tpu_sc as plsc

HIDDEN = 256
N_NODES = 10000
N_EDGES = 160000
_EPS = 1e-5

_NC = 2
_NS = 16
_NW = _NC * _NS
_B = 40


def _pre_body(x_ref, wa_ref, wb_ref, b_ref, ys_ref, yd_ref):
    xb = x_ref[...]
    ys_ref[...] = jnp.dot(xb, wa_ref[...], preferred_element_type=jnp.float32)
    yd_ref[...] = (
        jnp.dot(xb, wb_ref[...], preferred_element_type=jnp.float32) + b_ref[...]
    )


def _precompute(x, w0a, w0b, b0):
    nb = 1000
    return pl.pallas_call(
        _pre_body,
        grid=(N_NODES // nb,),
        in_specs=[
            pl.BlockSpec((nb, HIDDEN), lambda i: (i, 0)),
            pl.BlockSpec((HIDDEN, HIDDEN), lambda i: (0, 0)),
            pl.BlockSpec((HIDDEN, HIDDEN), lambda i: (0, 0)),
            pl.BlockSpec((1, HIDDEN), lambda i: (0, 0)),
        ],
        out_specs=[
            pl.BlockSpec((nb, HIDDEN), lambda i: (i, 0)),
            pl.BlockSpec((nb, HIDDEN), lambda i: (i, 0)),
        ],
        out_shape=[
            jax.ShapeDtypeStruct((N_NODES, HIDDEN), jnp.float32),
            jax.ShapeDtypeStruct((N_NODES, HIDDEN), jnp.float32),
        ],
    )(x, w0a, w0b, b0.reshape(1, HIDDEN))


def _gather_add(ys, yd, src, dst):
    per_w = N_EDGES // _NW
    nblk = per_w // _B
    mesh = plsc.VectorSubcoreMesh(core_axis_name="c", subcore_axis_name="s")

    @functools.partial(
        pl.kernel,
        mesh=mesh,
        out_type=jax.ShapeDtypeStruct((N_EDGES, HIDDEN), jnp.float32),
        scratch_types=[
            pltpu.VMEM((_B,), jnp.int32),
            pltpu.VMEM((_B,), jnp.int32),
            pltpu.VMEM((_B, HIDDEN), jnp.float32),
            pltpu.VMEM((_B, HIDDEN), jnp.float32),
            pltpu.SemaphoreType.DMA,
        ],
    )
    def k(ys_hbm, yd_hbm, src_hbm, dst_hbm, out_hbm, si, di, ra, rb, sem):
        wid = lax.axis_index("s") * _NC + lax.axis_index("c")
        base = wid * per_w

        def body(i, carry):
            off = base + i * _B
            pltpu.sync_copy(src_hbm.at[pl.ds(off, _B)], si)
            pltpu.sync_copy(dst_hbm.at[pl.ds(off, _B)], di)
            ca = pltpu.async_copy(ys_hbm.at[si], ra, sem)
            cb = pltpu.async_copy(yd_hbm.at[di], rb, sem)
            ca.wait()
            cb.wait()

            def addrow(r, c2):
                for j in range(HIDDEN // 16):
                    sl = pl.ds(j * 16, 16)
                    ra[r, sl] = ra[r, sl] + rb[r, sl]
                return c2

            lax.fori_loop(0, _B, addrow, 0)
            pltpu.sync_copy(ra, out_hbm.at[pl.ds(off, _B)])
            return carry

        lax.fori_loop(0, nblk, body, 0)

    return k(ys, yd, src, dst)


def _ln(z, g, b):
    mu = jnp.mean(z, axis=-1, keepdims=True)
    zc = z - mu
    var = jnp.mean(zc * zc, axis=-1, keepdims=True)
    return zc * lax.rsqrt(var + _EPS) * g + b


def _mlp_body(z_ref, w1_ref, b1_ref, w2_ref, b2_ref,
              g0_ref, be0_ref, g1_ref, be1_ref, g2_ref, be2_ref, out_ref):
    h = jnp.maximum(_ln(z_ref[...], g0_ref[...], be0_ref[...]), 0.0)
    h = jnp.dot(h, w1_ref[...], preferred_element_type=jnp.float32) + b1_ref[...]
    h = jnp.maximum(_ln(h, g1_ref[...], be1_ref[...]), 0.0)
    h = jnp.dot(h, w2_ref[...], preferred_element_type=jnp.float32) + b2_ref[...]
    out_ref[...] = jnp.tanh(_ln(h, g2_ref[...], be2_ref[...]))


def _mlp(z0, W1, b1, W2, b2, g0, be0, g1, be1, g2, be2):
    blk = 640
    vec = pl.BlockSpec((1, HIDDEN), lambda i: (0, 0))
    mat = pl.BlockSpec((HIDDEN, HIDDEN), lambda i: (0, 0))
    return pl.pallas_call(
        _mlp_body,
        grid=(N_EDGES // blk,),
        in_specs=[pl.BlockSpec((blk, HIDDEN), lambda i: (i, 0)),
                  mat, vec, mat, vec, vec, vec, vec, vec, vec, vec],
        out_specs=pl.BlockSpec((blk, HIDDEN), lambda i: (i, 0)),
        out_shape=jax.ShapeDtypeStruct((N_EDGES, HIDDEN), jnp.float32),
    )(z0, W1, b1.reshape(1, HIDDEN), W2, b2.reshape(1, HIDDEN),
      g0.reshape(1, HIDDEN), be0.reshape(1, HIDDEN),
      g1.reshape(1, HIDDEN), be1.reshape(1, HIDDEN),
      g2.reshape(1, HIDDEN), be2.reshape(1, HIDDEN))


def kernel(x, edge_index, W0, b0, g0, be0, W1, b1, g1, be1, W2, b2, g2, be2):
    src = edge_index[0].astype(jnp.int32)
    dst = edge_index[1].astype(jnp.int32)
    ys, yd = _precompute(x, W0[:HIDDEN], W0[HIDDEN:], b0)
    z0 = _gather_add(ys, yd, src, dst)
    return _mlp(z0, W1, b1, W2, b2, g0, be0, g1, be1, g2, be2)


# speedup vs baseline: 2.6152x; 1.3155x over previous
import functools

import jax
import jax.numpy as jnp
from jax import lax
from jax.experimental import pallas as pl
from jax.experimental.pallas import tpu as pltpu
from jax.experimental.pallas import tpu_sc as plsc

HIDDEN = 256
N_NODES = 10000
N_EDGES = 160000
_EPS = 1e-5

_NC = 2
_NS = 16
_NW = _NC * _NS
_B = 40


def _pre_body(x_ref, wa_ref, wb_ref, b_ref, ys_ref, yd_ref):
    xb = x_ref[...]
    ys_ref[...] = jnp.dot(xb, wa_ref[...], preferred_element_type=jnp.float32)
    yd_ref[...] = (
        jnp.dot(xb, wb_ref[...], preferred_element_type=jnp.float32) + b_ref[...]
    )


def _precompute(x, w0a, w0b, b0):
    nb = 1000
    return pl.pallas_call(
        _pre_body,
        grid=(N_NODES // nb,),
        in_specs=[
            pl.BlockSpec((nb, HIDDEN), lambda i: (i, 0)),
            pl.BlockSpec((HIDDEN, HIDDEN), lambda i: (0, 0)),
            pl.BlockSpec((HIDDEN, HIDDEN), lambda i: (0, 0)),
            pl.BlockSpec((1, HIDDEN), lambda i: (0, 0)),
        ],
        out_specs=[
            pl.BlockSpec((nb, HIDDEN), lambda i: (i, 0)),
            pl.BlockSpec((nb, HIDDEN), lambda i: (i, 0)),
        ],
        out_shape=[
            jax.ShapeDtypeStruct((N_NODES, HIDDEN), jnp.float32),
            jax.ShapeDtypeStruct((N_NODES, HIDDEN), jnp.float32),
        ],
    )(x, w0a, w0b, b0.reshape(1, HIDDEN))


_S = 4


def _gather2(ys, yd, src, dst):
    per_w = N_EDGES // _NW
    nblk = per_w // _B
    ngrp = nblk // _S
    mesh = plsc.VectorSubcoreMesh(core_axis_name="c", subcore_axis_name="s")

    @functools.partial(
        pl.kernel,
        mesh=mesh,
        out_type=[jax.ShapeDtypeStruct((N_EDGES, HIDDEN), jnp.float32),
                  jax.ShapeDtypeStruct((N_EDGES, HIDDEN), jnp.float32)],
        scratch_types=(
            [pltpu.VMEM((per_w,), jnp.int32)] * 2
            + [pltpu.VMEM((_B, HIDDEN), jnp.float32)] * (2 * _S)
            + [pltpu.SemaphoreType.DMA] * (4 * _S)
        ),
    )
    def k(ys_hbm, yd_hbm, src_hbm, dst_hbm, zs_hbm, zd_hbm, *rest):
        isv, idv = rest[0], rest[1]
        bufs = rest[2:2 + 2 * _S]
        gsem = rest[2 + 2 * _S:2 + 4 * _S]
        osem = rest[2 + 4 * _S:2 + 6 * _S]
        wid = lax.axis_index("s") * _NC + lax.axis_index("c")
        base = wid * per_w
        pltpu.sync_copy(src_hbm.at[pl.ds(base, per_w)], isv)
        pltpu.sync_copy(dst_hbm.at[pl.ds(base, per_w)], idv)

        tables = ((isv, ys_hbm, zs_hbm, 0), (idv, yd_hbm, zd_hbm, _S))

        def fire_gather(tbl, idxref, blk, s):
            pltpu.async_copy(tbl.at[idxref.at[pl.ds(blk * _B, _B)]], bufs[s],
                             gsem[s])

        def wait_gather(s):
            pltpu.make_async_copy(ys_hbm.at[pl.ds(0, _B)], bufs[s],
                                  gsem[s]).wait()

        def fire_out(outref, blk, s):
            pltpu.async_copy(bufs[s], outref.at[pl.ds(base + blk * _B, _B)],
                             osem[s])

        def wait_out(s):
            pltpu.make_async_copy(bufs[s], zs_hbm.at[pl.ds(0, _B)],
                                  osem[s]).wait()

        def group(g, carry):
            for idxref, tbl, _outref, s0 in tables:
                for j in range(_S):
                    s = s0 + j

                    @pl.when(g > 0)
                    def _w(s=s):
                        wait_out(s)

                    fire_gather(tbl, idxref, g * _S + j, s)
            for _idxref, _tbl, outref, s0 in tables:
                for j in range(_S):
                    s = s0 + j
                    wait_gather(s)
                    fire_out(outref, g * _S + j, s)
            return carry

        lax.fori_loop(0, ngrp, group, 0)

        for idxref, tbl, _outref, s0 in tables:
            wait_out(s0)
            fire_gather(tbl, idxref, nblk - 1, s0)
        for _idxref, _tbl, outref, s0 in tables:
            wait_gather(s0)
            fire_out(outref, nblk - 1, s0)
        for s in range(2 * _S):
            wait_out(s)

    return k(ys, yd, src, dst)


def _ln(z, g, b):
    mu = jnp.mean(z, axis=-1, keepdims=True)
    zc = z - mu
    var = jnp.mean(zc * zc, axis=-1, keepdims=True)
    return zc * lax.rsqrt(var + _EPS) * g + b


def _mlp_body(zs_ref, zd_ref, w1_ref, b1_ref, w2_ref, b2_ref,
              g0_ref, be0_ref, g1_ref, be1_ref, g2_ref, be2_ref, out_ref):
    z = zs_ref[...] + zd_ref[...]
    h = jnp.maximum(_ln(z, g0_ref[...], be0_ref[...]), 0.0)
    h = jnp.dot(h, w1_ref[...], preferred_element_type=jnp.float32) + b1_ref[...]
    h = jnp.maximum(_ln(h, g1_ref[...], be1_ref[...]), 0.0)
    h = jnp.dot(h, w2_ref[...], preferred_element_type=jnp.float32) + b2_ref[...]
    out_ref[...] = jnp.tanh(_ln(h, g2_ref[...], be2_ref[...]))


def _mlp(zs, zd, W1, b1, W2, b2, g0, be0, g1, be1, g2, be2):
    blk = 640
    vec = pl.BlockSpec((1, HIDDEN), lambda i: (0, 0))
    mat = pl.BlockSpec((HIDDEN, HIDDEN), lambda i: (0, 0))
    row = pl.BlockSpec((blk, HIDDEN), lambda i: (i, 0))
    return pl.pallas_call(
        _mlp_body,
        grid=(N_EDGES // blk,),
        in_specs=[row, row, mat, vec, mat, vec, vec, vec, vec, vec, vec, vec],
        out_specs=pl.BlockSpec((blk, HIDDEN), lambda i: (i, 0)),
        out_shape=jax.ShapeDtypeStruct((N_EDGES, HIDDEN), jnp.float32),
    )(zs, zd, W1, b1.reshape(1, HIDDEN), W2, b2.reshape(1, HIDDEN),
      g0.reshape(1, HIDDEN), be0.reshape(1, HIDDEN),
      g1.reshape(1, HIDDEN), be1.reshape(1, HIDDEN),
      g2.reshape(1, HIDDEN), be2.reshape(1, HIDDEN))


def kernel(x, edge_index, W0, b0, g0, be0, W1, b1, g1, be1, W2, b2, g2, be2):
    src = edge_index[0].astype(jnp.int32)
    dst = edge_index[1].astype(jnp.int32)
    ys, yd = _precompute(x, W0[:HIDDEN], W0[HIDDEN:], b0)
    zs, zd = _gather2(ys, yd, src, dst)
    return _mlp(zs, zd, W1, b1, W2, b2, g0, be0, g1, be1, g2, be2)
